# baseline (speedup 1.0000x reference)
import functools

import jax
import jax.numpy as jnp
from jax import lax
from jax.experimental import pallas as pl
from jax.experimental.pallas import tpu as pltpu
from jax.experimental.pallas import tpu_sc as plsc

N_ATOMS = 100000
FEATURE_DIM = 128
MAX_ROWS = 109

NUM_CORES = 2
NUM_SUBCORES = 16
NW = NUM_CORES * NUM_SUBCORES

CHUNK = 80
N_CHUNKS = N_ATOMS // CHUNK
BIG_CHUNKS = 40
SMALL_CHUNKS = 39
NBUF = 8

_mesh = plsc.VectorSubcoreMesh(core_axis_name="c", subcore_axis_name="s")


@functools.partial(
    pl.kernel,
    mesh=_mesh,
    out_type=jax.ShapeDtypeStruct((N_ATOMS, FEATURE_DIM), jnp.float32),
    scratch_types=[
        pltpu.VMEM((BIG_CHUNKS * CHUNK,), jnp.int32),
        *[pltpu.VMEM((CHUNK, FEATURE_DIM), jnp.float32)
          for _ in range(NBUF)],
        pltpu.VMEM_SHARED((MAX_ROWS, FEATURE_DIM), jnp.float32),
        *[pltpu.SemaphoreType.DMA for _ in range(2 * NBUF)],
    ],
)
def _embed_gather(idx_hbm, table_hbm, out_hbm, idx_v, *scr):
    rows = scr[:NBUF]
    shared_tab = scr[NBUF]
    gsem = scr[NBUF + 1:2 * NBUF + 1]
    osem = scr[2 * NBUF + 1:]
    sid = lax.axis_index("s")
    wid = sid * NUM_CORES + lax.axis_index("c")
    cbase = SMALL_CHUNKS * wid + jnp.minimum(wid, 2)

    @pl.when(sid == 0)
    def _():
        pltpu.sync_copy(table_hbm, shared_tab)

    @pl.when(wid < 2)
    def _():
        pltpu.sync_copy(idx_hbm.at[pl.ds(cbase * CHUNK, BIG_CHUNKS * CHUNK)],
                        idx_v)

    @pl.when(wid >= 2)
    def _():
        pltpu.sync_copy(idx_hbm.at[pl.ds(cbase * CHUNK, SMALL_CHUNKS * CHUNK)],
                        idx_v.at[pl.ds(0, SMALL_CHUNKS * CHUNK)])

    plsc.subcore_barrier()

    def start_gather(k, b):
        pltpu.async_copy(shared_tab.at[idx_v.at[pl.ds(k * CHUNK, CHUNK)]],
                         rows[b], gsem[b])

    def wait_gather(b):
        pltpu.make_async_copy(shared_tab.at[idx_v.at[pl.ds(0, CHUNK)]],
                              rows[b], gsem[b]).wait()

    def start_out(k, b):
        pltpu.async_copy(rows[b],
                         out_hbm.at[pl.ds((cbase + k) * CHUNK, CHUNK)],
                         osem[b])

    def wait_out(b):
        pltpu.make_async_copy(rows[b], out_hbm.at[pl.ds(0, CHUNK)],
                              osem[b]).wait()

    def run_chunks(nch):
        for b in range(NBUF):
            start_gather(b, b)

        def body(g, carry):
            for b in range(NBUF):
                k = NBUF * g + b
                wait_gather(b)
                start_out(k, b)

                @pl.when(k + NBUF < nch)
                def _():
                    wait_out(b)
                    start_gather(k + NBUF, b)

            return carry

        lax.fori_loop(0, nch // NBUF, body, 0)
        for t in range(nch % NBUF):
            k = (nch // NBUF) * NBUF + t
            wait_gather(k % NBUF)
            start_out(k, k % NBUF)
        for b in range(NBUF):
            wait_out(b)

    @pl.when(wid < 2)
    def _():
        run_chunks(BIG_CHUNKS)

    @pl.when(wid >= 2)
    def _():
        run_chunks(SMALL_CHUNKS)


def kernel(x, weight):
    return _embed_gather(x.astype(jnp.int32), weight)


# speedup vs baseline: 1.0009x; 1.0009x over previous
import functools

import jax
import jax.numpy as jnp
from jax import lax
from jax.experimental import pallas as pl
from jax.experimental.pallas import tpu as pltpu
from jax.experimental.pallas import tpu_sc as plsc

N_ATOMS = 100000
FEATURE_DIM = 128
MAX_ROWS = 109

NUM_CORES = 2
NUM_SUBCORES = 16
NW = NUM_CORES * NUM_SUBCORES

CHUNK = 80
N_CHUNKS = N_ATOMS // CHUNK
BIG_CHUNKS = 40
SMALL_CHUNKS = 39
NBUF = 4

_mesh = plsc.VectorSubcoreMesh(core_axis_name="c", subcore_axis_name="s")


@functools.partial(
    pl.kernel,
    mesh=_mesh,
    out_type=jax.ShapeDtypeStruct((N_ATOMS, FEATURE_DIM), jnp.float32),
    scratch_types=[
        pltpu.VMEM((BIG_CHUNKS * CHUNK,), jnp.int32),
        *[pltpu.VMEM((CHUNK, FEATURE_DIM), jnp.float32)
          for _ in range(NBUF)],
        pltpu.VMEM_SHARED((MAX_ROWS, FEATURE_DIM), jnp.float32),
        *[pltpu.SemaphoreType.DMA for _ in range(2 * NBUF)],
    ],
)
def _embed_gather(idx_hbm, table_hbm, out_hbm, idx_v, *scr):
    rows = scr[:NBUF]
    shared_tab = scr[NBUF]
    gsem = scr[NBUF + 1:2 * NBUF + 1]
    osem = scr[2 * NBUF + 1:]
    sid = lax.axis_index("s")
    wid = sid * NUM_CORES + lax.axis_index("c")
    cbase = SMALL_CHUNKS * wid + jnp.minimum(wid, 2)

    @pl.when(sid == 0)
    def _():
        pltpu.sync_copy(table_hbm, shared_tab)

    @pl.when(wid < 2)
    def _():
        pltpu.sync_copy(idx_hbm.at[pl.ds(cbase * CHUNK, BIG_CHUNKS * CHUNK)],
                        idx_v)

    @pl.when(wid >= 2)
    def _():
        pltpu.sync_copy(idx_hbm.at[pl.ds(cbase * CHUNK, SMALL_CHUNKS * CHUNK)],
                        idx_v.at[pl.ds(0, SMALL_CHUNKS * CHUNK)])

    plsc.subcore_barrier()

    def start_gather(k, b):
        pltpu.async_copy(shared_tab.at[idx_v.at[pl.ds(k * CHUNK, CHUNK)]],
                         rows[b], gsem[b])

    def wait_gather(b):
        pltpu.make_async_copy(shared_tab.at[idx_v.at[pl.ds(0, CHUNK)]],
                              rows[b], gsem[b]).wait()

    def start_out(k, b):
        pltpu.async_copy(rows[b],
                         out_hbm.at[pl.ds((cbase + k) * CHUNK, CHUNK)],
                         osem[b])

    def wait_out(b):
        pltpu.make_async_copy(rows[b], out_hbm.at[pl.ds(0, CHUNK)],
                              osem[b]).wait()

    def run_chunks(nch):
        for b in range(NBUF):
            start_gather(b, b)

        def body(g, carry):
            for b in range(NBUF):
                k = NBUF * g + b
                wait_gather(b)
                start_out(k, b)

                @pl.when(k + NBUF < nch)
                def _():
                    wait_out(b)
                    start_gather(k + NBUF, b)

            return carry

        lax.fori_loop(0, nch // NBUF, body, 0)
        for t in range(nch % NBUF):
            k = (nch // NBUF) * NBUF + t
            wait_gather(k % NBUF)
            start_out(k, k % NBUF)
        for b in range(NBUF):
            wait_out(b)

    @pl.when(wid < 2)
    def _():
        run_chunks(BIG_CHUNKS)

    @pl.when(wid >= 2)
    def _():
        run_chunks(SMALL_CHUNKS)


def kernel(x, weight):
    return _embed_gather(x.astype(jnp.int32), weight)
